# baseline (speedup 1.0000x reference)
import jax
import jax.numpy as jnp
import numpy as np
from jax.experimental import pallas as pl
from jax.experimental.pallas import tpu as pltpu

D = 2048
BC = 512


def _fused_kernel(x_ref, tail_ref,
                  w0f_ref, w0c_ref, w0v_ref,
                  w1f_ref, w1c_ref, w1v_ref,
                  w2f_ref, w2c_ref, w2v_ref,
                  bias_ref, pi_ref, v_ref):
    f32 = jnp.float32
    xb = x_ref[:, :].astype(jnp.bfloat16)
    gate = tail_ref[0:3, :]
    b = bias_ref[:, :]

    def mlp2(w0_ref, b0, w1_ref, b1):
        w0 = w0_ref[:, :].astype(jnp.bfloat16)
        h = jnp.tanh(jnp.dot(w0, xb, preferred_element_type=f32) + b0)
        h2 = jax.lax.dot_general(w1_ref[:, :], h, (((0,), (0,)), ((), ())),
                                 preferred_element_type=f32)
        return jnp.tanh(h2 + b1)

    hf = mlp2(w0f_ref, b[0:64], w1f_ref, b[192:256])
    hc = mlp2(w0c_ref, b[64:128], w1c_ref, b[256:320])
    hv = mlp2(w0v_ref, b[128:192], w1v_ref, b[320:384])

    of = jnp.dot(w2f_ref[:, :], hf, preferred_element_type=f32) + b[384:416]
    oc = jnp.dot(w2c_ref[:, :], hc, preferred_element_type=f32) + b[416:448]
    ov = jnp.dot(w2v_ref[:, :], hv, preferred_element_type=f32) + b[448:449]

    ones32 = jnp.ones((32, 32), f32)
    ef = jnp.exp(of)
    ec = jnp.exp(oc)
    sf = jnp.dot(ones32, ef, preferred_element_type=f32)
    sc = jnp.dot(ones32, ec, preferred_element_type=f32)
    mask = jnp.all(jnp.abs(gate) <= 0.1, axis=0, keepdims=True)
    pi_ref[0:32, :] = jnp.where(mask, ef / sf, 0.0)
    pi_ref[32:64, :] = jnp.where(mask, 0.0, ec / sc)
    v_ref[:, :] = ov


def kernel(observation, prev_action, prev_reward,
           pf_W0, pf_b0, pf_W1, pf_b1, pf_W2, pf_b2,
           pc_W0, pc_b0, pc_W1, pc_b1, pc_W2, pc_b2,
           v_W0, v_b0, v_W1, v_b1, v_W2, v_b2):
    B = observation.shape[0]
    f32 = jnp.float32

    obs_t = observation.T
    weights = [pf_W0.T, pc_W0.T, v_W0.T,
               pf_W1, pc_W1, v_W1,
               pf_W2.T, pc_W2.T, v_W2.T]
    bias = jnp.concatenate(
        [pf_b0, pc_b0, v_b0, pf_b1, pc_b1, v_b1,
         pf_b2, pc_b2, v_b2])[:, None]

    grid = (B // BC,)
    rep = lambda i: (0, 0)

    def wspec(arr):
        return pl.BlockSpec(arr.shape, rep)

    pi, v_t = pl.pallas_call(
        _fused_kernel,
        grid=grid,
        in_specs=[
            pl.BlockSpec((D, BC), lambda i: (0, i)),
            pl.BlockSpec((8, BC), lambda i: (D // 8, i)),
            *[wspec(w) for w in weights],
            pl.BlockSpec((449, 1), rep),
        ],
        out_specs=[
            pl.BlockSpec((64, BC), lambda i: (0, i)),
            pl.BlockSpec((1, BC), lambda i: (0, i)),
        ],
        out_shape=[
            jax.ShapeDtypeStruct((64, B), f32),
            jax.ShapeDtypeStruct((1, B), f32),
        ],
        compiler_params=pltpu.CompilerParams(
            dimension_semantics=("parallel",)),
    )(obs_t, obs_t, *weights, bias)
    return (pi.T, v_t[0])


# speedup vs baseline: 1.0976x; 1.0976x over previous
import jax
import jax.numpy as jnp
import numpy as np
from jax.experimental import pallas as pl
from jax.experimental.pallas import tpu as pltpu

D = 2048
BC = 2048


def _fused_kernel(x_ref, tail_ref,
                  w0f_ref, w0c_ref, w0v_ref,
                  w1f_ref, w1c_ref, w1v_ref,
                  w2f_ref, w2c_ref, w2v_ref,
                  bias_ref, pi_ref, v_ref):
    f32 = jnp.float32
    xb = x_ref[:, :].astype(jnp.bfloat16)
    gate = tail_ref[0:3, :]
    b = bias_ref[:, :]

    def mlp2(w0_ref, b0, w1_ref, b1):
        w0 = w0_ref[:, :].astype(jnp.bfloat16)
        h = jnp.tanh(jnp.dot(w0, xb, preferred_element_type=f32) + b0)
        h2 = jax.lax.dot_general(w1_ref[:, :], h, (((0,), (0,)), ((), ())),
                                 preferred_element_type=f32)
        return jnp.tanh(h2 + b1)

    hf = mlp2(w0f_ref, b[0:64], w1f_ref, b[192:256])
    hc = mlp2(w0c_ref, b[64:128], w1c_ref, b[256:320])
    hv = mlp2(w0v_ref, b[128:192], w1v_ref, b[320:384])

    of = jnp.dot(w2f_ref[:, :], hf, preferred_element_type=f32) + b[384:416]
    oc = jnp.dot(w2c_ref[:, :], hc, preferred_element_type=f32) + b[416:448]
    ov = jnp.dot(w2v_ref[:, :], hv, preferred_element_type=f32) + b[448:449]

    ones32 = jnp.ones((32, 32), f32)
    ef = jnp.exp(of)
    ec = jnp.exp(oc)
    sf = jnp.dot(ones32, ef, preferred_element_type=f32)
    sc = jnp.dot(ones32, ec, preferred_element_type=f32)
    mask = jnp.all(jnp.abs(gate) <= 0.1, axis=0, keepdims=True)
    pi_ref[0:32, :] = jnp.where(mask, ef / sf, 0.0)
    pi_ref[32:64, :] = jnp.where(mask, 0.0, ec / sc)
    v_ref[:, :] = ov


def kernel(observation, prev_action, prev_reward,
           pf_W0, pf_b0, pf_W1, pf_b1, pf_W2, pf_b2,
           pc_W0, pc_b0, pc_W1, pc_b1, pc_W2, pc_b2,
           v_W0, v_b0, v_W1, v_b1, v_W2, v_b2):
    B = observation.shape[0]
    f32 = jnp.float32

    obs_t = observation.T
    weights = [pf_W0.T, pc_W0.T, v_W0.T,
               pf_W1, pc_W1, v_W1,
               pf_W2.T, pc_W2.T, v_W2.T]
    bias = jnp.concatenate(
        [pf_b0, pc_b0, v_b0, pf_b1, pc_b1, v_b1,
         pf_b2, pc_b2, v_b2])[:, None]

    grid = (B // BC,)
    rep = lambda i: (0, 0)

    def wspec(arr):
        return pl.BlockSpec(arr.shape, rep)

    pi, v_t = pl.pallas_call(
        _fused_kernel,
        grid=grid,
        in_specs=[
            pl.BlockSpec((D, BC), lambda i: (0, i)),
            pl.BlockSpec((8, BC), lambda i: (D // 8, i)),
            *[wspec(w) for w in weights],
            pl.BlockSpec((449, 1), rep),
        ],
        out_specs=[
            pl.BlockSpec((64, BC), lambda i: (0, i)),
            pl.BlockSpec((1, BC), lambda i: (0, i)),
        ],
        out_shape=[
            jax.ShapeDtypeStruct((64, B), f32),
            jax.ShapeDtypeStruct((1, B), f32),
        ],
        compiler_params=pltpu.CompilerParams(
            dimension_semantics=("parallel",)),
    )(obs_t, obs_t, *weights, bias)
    return (pi.T, v_t[0])


# speedup vs baseline: 1.1131x; 1.0141x over previous
import jax
import jax.numpy as jnp
import numpy as np
from jax.experimental import pallas as pl
from jax.experimental.pallas import tpu as pltpu

D = 2048
BC = 1024


def _fused_kernel(x_ref, tail_ref,
                  w0f_ref, w0c_ref, w0v_ref,
                  w1f_ref, w1c_ref, w1v_ref,
                  w2f_ref, w2c_ref, w2v_ref,
                  bias_ref, pi_ref, v_ref):
    f32 = jnp.float32
    xb = x_ref[:, :].astype(jnp.bfloat16)
    gate = tail_ref[0:3, :]
    b = bias_ref[:, :]

    def mlp2(w0_ref, b0, w1_ref, b1):
        w0 = w0_ref[:, :].astype(jnp.bfloat16)
        h = jnp.tanh(jnp.dot(w0, xb, preferred_element_type=f32) + b0)
        h2 = jax.lax.dot_general(w1_ref[:, :], h, (((0,), (0,)), ((), ())),
                                 preferred_element_type=f32)
        return jnp.tanh(h2 + b1)

    hf = mlp2(w0f_ref, b[0:64], w1f_ref, b[192:256])
    hc = mlp2(w0c_ref, b[64:128], w1c_ref, b[256:320])
    hv = mlp2(w0v_ref, b[128:192], w1v_ref, b[320:384])

    of = jnp.dot(w2f_ref[:, :], hf, preferred_element_type=f32) + b[384:416]
    oc = jnp.dot(w2c_ref[:, :], hc, preferred_element_type=f32) + b[416:448]
    ov = jnp.dot(w2v_ref[:, :], hv, preferred_element_type=f32) + b[448:449]

    ones32 = jnp.ones((32, 32), f32)
    ef = jnp.exp(of)
    ec = jnp.exp(oc)
    sf = jnp.dot(ones32, ef, preferred_element_type=f32)
    sc = jnp.dot(ones32, ec, preferred_element_type=f32)
    mask = jnp.all(jnp.abs(gate) <= 0.1, axis=0, keepdims=True)
    pi_ref[0:32, :] = jnp.where(mask, ef / sf, 0.0)
    pi_ref[32:64, :] = jnp.where(mask, 0.0, ec / sc)
    v_ref[:, :] = ov


def kernel(observation, prev_action, prev_reward,
           pf_W0, pf_b0, pf_W1, pf_b1, pf_W2, pf_b2,
           pc_W0, pc_b0, pc_W1, pc_b1, pc_W2, pc_b2,
           v_W0, v_b0, v_W1, v_b1, v_W2, v_b2):
    B = observation.shape[0]
    f32 = jnp.float32

    obs_t = observation.T
    weights = [pf_W0.T, pc_W0.T, v_W0.T,
               pf_W1, pc_W1, v_W1,
               pf_W2.T, pc_W2.T, v_W2.T]
    bias = jnp.concatenate(
        [pf_b0, pc_b0, v_b0, pf_b1, pc_b1, v_b1,
         pf_b2, pc_b2, v_b2])[:, None]

    grid = (B // BC,)
    rep = lambda i: (0, 0)

    def wspec(arr):
        return pl.BlockSpec(arr.shape, rep)

    pi, v_t = pl.pallas_call(
        _fused_kernel,
        grid=grid,
        in_specs=[
            pl.BlockSpec((D, BC), lambda i: (0, i)),
            pl.BlockSpec((8, BC), lambda i: (D // 8, i)),
            *[wspec(w) for w in weights],
            pl.BlockSpec((449, 1), rep),
        ],
        out_specs=[
            pl.BlockSpec((64, BC), lambda i: (0, i)),
            pl.BlockSpec((1, BC), lambda i: (0, i)),
        ],
        out_shape=[
            jax.ShapeDtypeStruct((64, B), f32),
            jax.ShapeDtypeStruct((1, B), f32),
        ],
        compiler_params=pltpu.CompilerParams(
            dimension_semantics=("parallel",)),
    )(obs_t, obs_t, *weights, bias)
    return (pi.T, v_t[0])
